# baseline (speedup 1.0000x reference)
import functools

import jax
import jax.numpy as jnp
from jax import lax
from jax.experimental import pallas as pl
from jax.experimental.pallas import tpu as pltpu
from jax.experimental.pallas import tpu_sc as plsc

N = 10000
E = 320000
D = 128

NC = 2
NS = 16
EB = 80
NBT = E // EB
NBW = NBT // (NC * NS)
ROWS_PER_SUB = 624
TAIL_START = ROWS_PER_SUB * NS
TAIL_ROWS = N - TAIL_START
NR = 4
GL = 2
NI = 8
GRP = 8
NGRP = NBW // GRP
TAILJ = NGRP * GRP


def _relu_body(x_ref, h_ref):
    h_ref[...] = jnp.maximum(x_ref[...], 0.0)


def _final_body(h_ref, p_ref, w_ref, b_ref, g_ref, be_ref, o_ref):
    pre = h_ref[...] + p_ref[0] + p_ref[1]
    out = jnp.dot(pre, w_ref[...].T, preferred_element_type=jnp.float32)
    out = out + b_ref[...]
    mean = jnp.mean(out, axis=0, keepdims=True)
    var = jnp.mean((out - mean) ** 2, axis=0, keepdims=True)
    o_ref[...] = (out - mean) * lax.rsqrt(var + 1e-5) * g_ref[...] + be_ref[...]


def _sc_segment_sum(h, eidx3, zeros):
    mesh = plsc.VectorSubcoreMesh(core_axis_name="c", subcore_axis_name="s")

    @functools.partial(
        pl.kernel,
        out_type=jax.ShapeDtypeStruct((NC, N, D), jnp.float32),
        mesh=mesh,
        scratch_types=[
            pltpu.VMEM((NI, 2, EB), jnp.int32),
            pltpu.VMEM((NR, EB, D), jnp.float32),
            pltpu.VMEM_SHARED((N, D), jnp.float32),
            [pltpu.SemaphoreType.DMA] * NI,
            [pltpu.SemaphoreType.DMA] * NR,
            [pltpu.SemaphoreType.DMA] * NR,
            pltpu.SemaphoreType.DMA,
        ],
    )
    def seg_sum(h_hbm, eidx_hbm, zeros_hbm, out_hbm,
                idx_v, rows_v, agg_sh, isems, gsems, ssems, zsem):
        c = lax.axis_index("c")
        s = lax.axis_index("s")
        b0 = (c * NS + s) * NBW

        def start_idx(j, islot):
            return pltpu.async_copy(eidx_hbm.at[b0 + j], idx_v.at[islot],
                                    isems[islot])

        def wait_idx(islot):
            pltpu.make_async_copy(eidx_hbm.at[b0], idx_v.at[islot],
                                  isems[islot]).wait()

        def start_gather(islot, buf):
            return pltpu.async_copy(h_hbm.at[idx_v.at[islot, 0]],
                                    rows_v.at[buf], gsems[buf])

        def wait_gather(islot, buf):
            pltpu.make_async_copy(h_hbm.at[idx_v.at[islot, 0]],
                                  rows_v.at[buf], gsems[buf]).wait()

        def start_scatter(islot, buf):
            return pltpu.async_copy(rows_v.at[buf],
                                    agg_sh.at[idx_v.at[islot, 1]],
                                    ssems[buf], add=True)

        def wait_scatter(islot, buf):
            pltpu.make_async_copy(rows_v.at[buf],
                                  agg_sh.at[idx_v.at[islot, 1]],
                                  ssems[buf]).wait()

        pltpu.async_copy(zeros_hbm.at[pl.ds(s * ROWS_PER_SUB, ROWS_PER_SUB)],
                         agg_sh.at[pl.ds(s * ROWS_PER_SUB, ROWS_PER_SUB)],
                         zsem)

        @pl.when(s == NS - 1)
        def _zero_tail():
            pltpu.async_copy(zeros_hbm.at[pl.ds(TAIL_START, TAIL_ROWS)],
                             agg_sh.at[pl.ds(TAIL_START, TAIL_ROWS)], zsem)

        for k in range(NI):
            start_idx(k, k)
        for k in range(GL):
            wait_idx(k)
            start_gather(k, k)

        pltpu.make_async_copy(
            zeros_hbm.at[pl.ds(s * ROWS_PER_SUB, ROWS_PER_SUB)],
            agg_sh.at[pl.ds(s * ROWS_PER_SUB, ROWS_PER_SUB)], zsem).wait()

        @pl.when(s == NS - 1)
        def _zero_tail_wait():
            pltpu.make_async_copy(zeros_hbm.at[pl.ds(TAIL_START, TAIL_ROWS)],
                                  agg_sh.at[pl.ds(TAIL_START, TAIL_ROWS)],
                                  zsem).wait()
        plsc.subcore_barrier()

        def step(t, k):
            b = k % NR
            wait_gather(k, b)
            start_scatter(k, b)

            @pl.when(t >= GL)
            def _after():
                wait_scatter((k - GL) % NI, (k - GL) % NR)

                @pl.when(t + NI - GL < NBW)
                def _prefetch_idx():
                    start_idx(t + NI - GL, (k - GL) % NI)

            @pl.when(t + GL < NBW)
            def _refill():
                wait_idx((k + GL) % NI)
                start_gather((k + GL) % NI, (k + GL) % NR)

        def body(g, carry):
            t0 = g * GRP
            for k in range(GRP):
                step(t0 + k, k)
            return carry

        lax.fori_loop(0, NGRP, body, 0, unroll=False)

        for t in range(TAILJ, NBW):
            step(t, t % NI)

        for t in range(NBW - GL, NBW):
            wait_scatter(t % NI, t % NR)

        plsc.subcore_barrier()
        pltpu.sync_copy(agg_sh.at[pl.ds(s * ROWS_PER_SUB, ROWS_PER_SUB)],
                        out_hbm.at[c, pl.ds(s * ROWS_PER_SUB, ROWS_PER_SUB)])

        @pl.when(s == NS - 1)
        def _flush_tail():
            pltpu.sync_copy(agg_sh.at[pl.ds(TAIL_START, TAIL_ROWS)],
                            out_hbm.at[c, pl.ds(TAIL_START, TAIL_ROWS)])

    return seg_sum(h, eidx3, zeros)


def kernel(x, edge_index, W, b, gamma, beta):
    h = pl.pallas_call(
        _relu_body,
        out_shape=jax.ShapeDtypeStruct((N, D), jnp.float32),
    )(x)

    eidx3 = edge_index.reshape(2, NBT, EB).transpose(1, 0, 2)
    zeros = jnp.zeros((N, D), jnp.float32)
    parts = _sc_segment_sum(h, eidx3, zeros)

    out = pl.pallas_call(
        _final_body,
        out_shape=jax.ShapeDtypeStruct((N, D), jnp.float32),
    )(h, parts, W, b.reshape(1, D), gamma.reshape(1, D), beta.reshape(1, D))
    return out


# speedup vs baseline: 1.0940x; 1.0940x over previous
import functools

import jax
import jax.numpy as jnp
from jax import lax
from jax.experimental import pallas as pl
from jax.experimental.pallas import tpu as pltpu
from jax.experimental.pallas import tpu_sc as plsc

N = 10000
E = 320000
D = 128

NC = 2
NS = 16
EB = 125
NBT = E // EB
NBW = NBT // (NC * NS)
ROWS_PER_SUB = 624
TAIL_START = ROWS_PER_SUB * NS
TAIL_ROWS = N - TAIL_START
NR = 3
GL = 2
SLACK = NR - GL
NI = 6
GRP = 6
NGRP = NBW // GRP
TAILJ = NGRP * GRP


def _relu_body(x_ref, h_ref):
    h_ref[...] = jnp.maximum(x_ref[...], 0.0)


def _final_body(h_ref, p_ref, w_ref, b_ref, g_ref, be_ref, o_ref):
    pre = h_ref[...] + p_ref[0] + p_ref[1]
    out = jnp.dot(pre, w_ref[...].T, preferred_element_type=jnp.float32)
    out = out + b_ref[...]
    mean = jnp.mean(out, axis=0, keepdims=True)
    var = jnp.mean((out - mean) ** 2, axis=0, keepdims=True)
    o_ref[...] = (out - mean) * lax.rsqrt(var + 1e-5) * g_ref[...] + be_ref[...]


def _sc_segment_sum(h, eidx3, zeros):
    mesh = plsc.VectorSubcoreMesh(core_axis_name="c", subcore_axis_name="s")

    @functools.partial(
        pl.kernel,
        out_type=jax.ShapeDtypeStruct((NC, N, D), jnp.float32),
        mesh=mesh,
        scratch_types=[
            pltpu.VMEM((NI, 2, EB), jnp.int32),
            pltpu.VMEM((NR, EB, D), jnp.float32),
            pltpu.VMEM_SHARED((N, D), jnp.float32),
            [pltpu.SemaphoreType.DMA] * NI,
            [pltpu.SemaphoreType.DMA] * NR,
            [pltpu.SemaphoreType.DMA] * NR,
            pltpu.SemaphoreType.DMA,
        ],
    )
    def seg_sum(h_hbm, eidx_hbm, zeros_hbm, out_hbm,
                idx_v, rows_v, agg_sh, isems, gsems, ssems, zsem):
        c = lax.axis_index("c")
        s = lax.axis_index("s")
        b0 = (c * NS + s) * NBW

        def start_idx(j, islot):
            return pltpu.async_copy(eidx_hbm.at[b0 + j], idx_v.at[islot],
                                    isems[islot])

        def wait_idx(islot):
            pltpu.make_async_copy(eidx_hbm.at[b0], idx_v.at[islot],
                                  isems[islot]).wait()

        def start_gather(islot, buf):
            return pltpu.async_copy(h_hbm.at[idx_v.at[islot, 0]],
                                    rows_v.at[buf], gsems[buf])

        def wait_gather(islot, buf):
            pltpu.make_async_copy(h_hbm.at[idx_v.at[islot, 0]],
                                  rows_v.at[buf], gsems[buf]).wait()

        def start_scatter(islot, buf):
            return pltpu.async_copy(rows_v.at[buf],
                                    agg_sh.at[idx_v.at[islot, 1]],
                                    ssems[buf], add=True)

        def wait_scatter(islot, buf):
            pltpu.make_async_copy(rows_v.at[buf],
                                  agg_sh.at[idx_v.at[islot, 1]],
                                  ssems[buf]).wait()

        pltpu.async_copy(zeros_hbm.at[pl.ds(s * ROWS_PER_SUB, ROWS_PER_SUB)],
                         agg_sh.at[pl.ds(s * ROWS_PER_SUB, ROWS_PER_SUB)],
                         zsem)

        @pl.when(s == NS - 1)
        def _zero_tail():
            pltpu.async_copy(zeros_hbm.at[pl.ds(TAIL_START, TAIL_ROWS)],
                             agg_sh.at[pl.ds(TAIL_START, TAIL_ROWS)], zsem)

        for k in range(NI):
            start_idx(k, k)
        for k in range(GL):
            wait_idx(k)
            start_gather(k, k)

        pltpu.make_async_copy(
            zeros_hbm.at[pl.ds(s * ROWS_PER_SUB, ROWS_PER_SUB)],
            agg_sh.at[pl.ds(s * ROWS_PER_SUB, ROWS_PER_SUB)], zsem).wait()

        @pl.when(s == NS - 1)
        def _zero_tail_wait():
            pltpu.make_async_copy(zeros_hbm.at[pl.ds(TAIL_START, TAIL_ROWS)],
                                  agg_sh.at[pl.ds(TAIL_START, TAIL_ROWS)],
                                  zsem).wait()
        plsc.subcore_barrier()

        def step(t, k):
            b = k % NR
            wait_gather(k, b)
            start_scatter(k, b)

            @pl.when(t >= SLACK)
            def _after():
                wait_scatter((k - SLACK) % NI, (k - SLACK) % NR)

                @pl.when(t + NI - SLACK < NBW)
                def _prefetch_idx():
                    start_idx(t + NI - SLACK, (k - SLACK) % NI)

            @pl.when(t + GL < NBW)
            def _refill():
                wait_idx((k + GL) % NI)
                start_gather((k + GL) % NI, (k + GL) % NR)

        def body(g, carry):
            t0 = g * GRP
            for k in range(GRP):
                step(t0 + k, k)
            return carry

        lax.fori_loop(0, NGRP, body, 0, unroll=False)

        for t in range(TAILJ, NBW):
            step(t, t % NI)

        for t in range(NBW - SLACK, NBW):
            wait_scatter(t % NI, t % NR)

        plsc.subcore_barrier()
        pltpu.sync_copy(agg_sh.at[pl.ds(s * ROWS_PER_SUB, ROWS_PER_SUB)],
                        out_hbm.at[c, pl.ds(s * ROWS_PER_SUB, ROWS_PER_SUB)])

        @pl.when(s == NS - 1)
        def _flush_tail():
            pltpu.sync_copy(agg_sh.at[pl.ds(TAIL_START, TAIL_ROWS)],
                            out_hbm.at[c, pl.ds(TAIL_START, TAIL_ROWS)])

    return seg_sum(h, eidx3, zeros)


def kernel(x, edge_index, W, b, gamma, beta):
    h = pl.pallas_call(
        _relu_body,
        out_shape=jax.ShapeDtypeStruct((N, D), jnp.float32),
    )(x)

    eidx3 = edge_index.reshape(2, NBT, EB).transpose(1, 0, 2)
    zeros = jnp.zeros((N, D), jnp.float32)
    parts = _sc_segment_sum(h, eidx3, zeros)

    out = pl.pallas_call(
        _final_body,
        out_shape=jax.ShapeDtypeStruct((N, D), jnp.float32),
    )(h, parts, W, b.reshape(1, D), gamma.reshape(1, D), beta.reshape(1, D))
    return out


# speedup vs baseline: 1.1684x; 1.0680x over previous
import functools

import jax
import jax.numpy as jnp
from jax import lax
from jax.experimental import pallas as pl
from jax.experimental.pallas import tpu as pltpu
from jax.experimental.pallas import tpu_sc as plsc

N = 10000
E = 320000
D = 128

NC = 2
NS = 16
EB = 125
NBT = E // EB
NBW = NBT // (NC * NS)
ROWS_PER_SUB = 624
TAIL_START = ROWS_PER_SUB * NS
TAIL_ROWS = N - TAIL_START
NBUF = 3
NI = 2 * NBUF
GRP = NI
NGRP = NBW // GRP
TAILJ = NGRP * GRP


def _relu_body(x_ref, h_ref):
    h_ref[...] = jnp.maximum(x_ref[...], 0.0)


def _final_body(h_ref, p_ref, w_ref, b_ref, g_ref, be_ref, o_ref):
    pre = h_ref[...] + p_ref[0] + p_ref[1]
    out = jnp.dot(pre, w_ref[...].T, preferred_element_type=jnp.float32)
    out = out + b_ref[...]
    mean = jnp.mean(out, axis=0, keepdims=True)
    var = jnp.mean((out - mean) ** 2, axis=0, keepdims=True)
    o_ref[...] = (out - mean) * lax.rsqrt(var + 1e-5) * g_ref[...] + be_ref[...]


def _sc_segment_sum(h, eidx3, zeros):
    mesh = plsc.VectorSubcoreMesh(core_axis_name="c", subcore_axis_name="s")

    @functools.partial(
        pl.kernel,
        out_type=jax.ShapeDtypeStruct((NC, N, D), jnp.float32),
        mesh=mesh,
        scratch_types=[
            pltpu.VMEM((NI, 2, EB), jnp.int32),
            pltpu.VMEM((NBUF, EB, D), jnp.float32),
            pltpu.VMEM_SHARED((N, D), jnp.float32),
            [pltpu.SemaphoreType.DMA] * NI,
            [pltpu.SemaphoreType.DMA] * NBUF,
            pltpu.SemaphoreType.DMA,
        ],
    )
    def seg_sum(h_hbm, eidx_hbm, zeros_hbm, out_hbm,
                idx_v, rows_v, agg_sh, isems, gsems, zsem):
        c = lax.axis_index("c")
        s = lax.axis_index("s")
        b0 = (c * NS + s) * NBW
        pltpu.async_copy(zeros_hbm.at[pl.ds(s * ROWS_PER_SUB, ROWS_PER_SUB)],
                         agg_sh.at[pl.ds(s * ROWS_PER_SUB, ROWS_PER_SUB)],
                         zsem)

        @pl.when(s == NS - 1)
        def _zero_tail():
            pltpu.async_copy(zeros_hbm.at[pl.ds(TAIL_START, TAIL_ROWS)],
                             agg_sh.at[pl.ds(TAIL_START, TAIL_ROWS)], zsem)

        def start_idx(j, islot):
            return pltpu.async_copy(eidx_hbm.at[b0 + j], idx_v.at[islot],
                                    isems[islot])

        def start_gather(islot, buf):
            return pltpu.async_copy(h_hbm.at[idx_v.at[islot, 0]],
                                    rows_v.at[buf], gsems[buf])

        def wait_idx(j, islot):
            pltpu.make_async_copy(eidx_hbm.at[b0 + j], idx_v.at[islot],
                                  isems[islot]).wait()

        def wait_gather(islot, buf):
            pltpu.make_async_copy(h_hbm.at[idx_v.at[islot, 0]],
                                  rows_v.at[buf], gsems[buf]).wait()

        def scatter(islot, buf):
            pltpu.sync_copy(rows_v.at[buf], agg_sh.at[idx_v.at[islot, 1]],
                            add=True)

        for k in range(NI):
            start_idx(k, k)
        for k in range(NBUF):
            wait_idx(k, k)
            start_gather(k, k)

        pltpu.make_async_copy(
            zeros_hbm.at[pl.ds(s * ROWS_PER_SUB, ROWS_PER_SUB)],
            agg_sh.at[pl.ds(s * ROWS_PER_SUB, ROWS_PER_SUB)], zsem).wait()

        @pl.when(s == NS - 1)
        def _zero_tail_wait():
            pltpu.make_async_copy(zeros_hbm.at[pl.ds(TAIL_START, TAIL_ROWS)],
                                  agg_sh.at[pl.ds(TAIL_START, TAIL_ROWS)],
                                  zsem).wait()
        plsc.subcore_barrier()

        def body(g, carry):
            j0 = g * GRP
            for k in range(GRP):
                j = j0 + k
                buf = k % NBUF
                wait_gather(k, buf)
                scatter(k, buf)

                @pl.when(j + NI < NBW)
                def _prefetch_idx():
                    start_idx(j + NI, k)

                @pl.when(j + NBUF < NBW)
                def _refill():
                    kn = (k + NBUF) % NI
                    wait_idx(j + NBUF, kn)
                    start_gather(kn, buf)
            return carry

        lax.fori_loop(0, NGRP, body, 0, unroll=False)

        for j in range(TAILJ, NBW):
            k = j % NI
            buf = j % NBUF
            wait_gather(k, buf)
            scatter(k, buf)

        plsc.subcore_barrier()
        pltpu.sync_copy(agg_sh.at[pl.ds(s * ROWS_PER_SUB, ROWS_PER_SUB)],
                        out_hbm.at[c, pl.ds(s * ROWS_PER_SUB, ROWS_PER_SUB)])

        @pl.when(s == NS - 1)
        def _flush_tail():
            pltpu.sync_copy(agg_sh.at[pl.ds(TAIL_START, TAIL_ROWS)],
                            out_hbm.at[c, pl.ds(TAIL_START, TAIL_ROWS)])

    return seg_sum(h, eidx3, zeros)


def kernel(x, edge_index, W, b, gamma, beta):
    h = pl.pallas_call(
        _relu_body,
        out_shape=jax.ShapeDtypeStruct((N, D), jnp.float32),
    )(x)

    eidx3 = edge_index.reshape(2, NBT, EB).transpose(1, 0, 2)
    zeros = jnp.zeros((N, D), jnp.float32)
    parts = _sc_segment_sum(h, eidx3, zeros)

    out = pl.pallas_call(
        _final_body,
        out_shape=jax.ShapeDtypeStruct((N, D), jnp.float32),
    )(h, parts, W, b.reshape(1, D), gamma.reshape(1, D), beta.reshape(1, D))
    return out


# speedup vs baseline: 1.2894x; 1.1035x over previous
import functools

import jax
import jax.numpy as jnp
from jax import lax
from jax.experimental import pallas as pl
from jax.experimental.pallas import tpu as pltpu
from jax.experimental.pallas import tpu_sc as plsc

N = 10000
E = 320000
D = 128

NC = 2
NS = 16
EB = 128
NBT = E // EB
NW = NC * NS
NBW = NBT // NW
NXTRA = NBT - NBW * NW
ROWS_PER_SUB = 624
TAIL_START = ROWS_PER_SUB * NS
TAIL_ROWS = N - TAIL_START
NBUF = 3
NI = 2 * NBUF
GRP = NI
NGRP = NBW // GRP
TAILJ = NGRP * GRP


def _relu_body(x_ref, h_ref):
    h_ref[...] = jnp.maximum(x_ref[...], 0.0)


def _final_body(h_ref, p_ref, w_ref, b_ref, g_ref, be_ref, o_ref):
    pre = h_ref[...] + p_ref[0] + p_ref[1]
    out = jnp.dot(pre, w_ref[...].T, preferred_element_type=jnp.float32)
    out = out + b_ref[...]
    mean = jnp.mean(out, axis=0, keepdims=True)
    var = jnp.mean((out - mean) ** 2, axis=0, keepdims=True)
    o_ref[...] = (out - mean) * lax.rsqrt(var + 1e-5) * g_ref[...] + be_ref[...]


def _sc_segment_sum(h, eflat, zeros):
    mesh = plsc.VectorSubcoreMesh(core_axis_name="c", subcore_axis_name="s")

    @functools.partial(
        pl.kernel,
        out_type=jax.ShapeDtypeStruct((NC, N, D), jnp.float32),
        mesh=mesh,
        scratch_types=[
            pltpu.VMEM((NI, 2, EB), jnp.int32),
            pltpu.VMEM((NBUF, EB, D), jnp.float32),
            pltpu.VMEM_SHARED((N, D), jnp.float32),
            [pltpu.SemaphoreType.DMA] * NI,
            [pltpu.SemaphoreType.DMA] * NBUF,
            pltpu.SemaphoreType.DMA,
        ],
    )
    def seg_sum(h_hbm, eidx_hbm, zeros_hbm, out_hbm,
                idx_v, rows_v, agg_sh, isems, gsems, zsem):
        c = lax.axis_index("c")
        s = lax.axis_index("s")
        w = c * NS + s
        b0 = w * NBW + jnp.minimum(w, NXTRA)
        nbw = NBW + jnp.where(w < NXTRA, 1, 0)
        pltpu.async_copy(zeros_hbm.at[pl.ds(s * ROWS_PER_SUB, ROWS_PER_SUB)],
                         agg_sh.at[pl.ds(s * ROWS_PER_SUB, ROWS_PER_SUB)],
                         zsem)

        @pl.when(s == NS - 1)
        def _zero_tail():
            pltpu.async_copy(zeros_hbm.at[pl.ds(TAIL_START, TAIL_ROWS)],
                             agg_sh.at[pl.ds(TAIL_START, TAIL_ROWS)], zsem)

        def start_idx(j, islot):
            off = (b0 + j) * EB
            pltpu.async_copy(eidx_hbm.at[pl.ds(off, EB)],
                             idx_v.at[islot, 0], isems[islot])
            pltpu.async_copy(eidx_hbm.at[pl.ds(E + off, EB)],
                             idx_v.at[islot, 1], isems[islot])

        def start_gather(islot, buf):
            return pltpu.async_copy(h_hbm.at[idx_v.at[islot, 0]],
                                    rows_v.at[buf], gsems[buf])

        def wait_idx(j, islot):
            off = (b0 + j) * EB
            pltpu.make_async_copy(eidx_hbm.at[pl.ds(off, EB)],
                                  idx_v.at[islot, 0], isems[islot]).wait()
            pltpu.make_async_copy(eidx_hbm.at[pl.ds(E + off, EB)],
                                  idx_v.at[islot, 1], isems[islot]).wait()

        def wait_gather(islot, buf):
            pltpu.make_async_copy(h_hbm.at[idx_v.at[islot, 0]],
                                  rows_v.at[buf], gsems[buf]).wait()

        def scatter(islot, buf):
            pltpu.sync_copy(rows_v.at[buf], agg_sh.at[idx_v.at[islot, 1]],
                            add=True)

        for k in range(NI):
            start_idx(k, k)
        for k in range(NBUF):
            wait_idx(k, k)
            start_gather(k, k)

        pltpu.make_async_copy(
            zeros_hbm.at[pl.ds(s * ROWS_PER_SUB, ROWS_PER_SUB)],
            agg_sh.at[pl.ds(s * ROWS_PER_SUB, ROWS_PER_SUB)], zsem).wait()

        @pl.when(s == NS - 1)
        def _zero_tail_wait():
            pltpu.make_async_copy(zeros_hbm.at[pl.ds(TAIL_START, TAIL_ROWS)],
                                  agg_sh.at[pl.ds(TAIL_START, TAIL_ROWS)],
                                  zsem).wait()
        plsc.subcore_barrier()

        def body(g, carry):
            j0 = g * GRP
            for k in range(GRP):
                j = j0 + k
                buf = k % NBUF
                wait_gather(k, buf)
                scatter(k, buf)

                @pl.when(j + NI < nbw)
                def _prefetch_idx():
                    start_idx(j + NI, k)

                @pl.when(j + NBUF < nbw)
                def _refill():
                    kn = (k + NBUF) % NI
                    wait_idx(j + NBUF, kn)
                    start_gather(kn, buf)
            return carry

        lax.fori_loop(0, NGRP, body, 0, unroll=False)

        for j in range(TAILJ, TAILJ + (1 if NXTRA else 0)):
            k = j % NI
            buf = j % NBUF

            @pl.when(w < NXTRA)
            def _extra():
                wait_gather(k, buf)
                scatter(k, buf)

        plsc.subcore_barrier()
        pltpu.sync_copy(agg_sh.at[pl.ds(s * ROWS_PER_SUB, ROWS_PER_SUB)],
                        out_hbm.at[c, pl.ds(s * ROWS_PER_SUB, ROWS_PER_SUB)])

        @pl.when(s == NS - 1)
        def _flush_tail():
            pltpu.sync_copy(agg_sh.at[pl.ds(TAIL_START, TAIL_ROWS)],
                            out_hbm.at[c, pl.ds(TAIL_START, TAIL_ROWS)])

    return seg_sum(h, eflat, zeros)


def kernel(x, edge_index, W, b, gamma, beta):
    h = pl.pallas_call(
        _relu_body,
        out_shape=jax.ShapeDtypeStruct((N, D), jnp.float32),
    )(x)

    eflat = edge_index.reshape(2 * E)
    zeros = jnp.zeros((N, D), jnp.float32)
    parts = _sc_segment_sum(h, eflat, zeros)

    out = pl.pallas_call(
        _final_body,
        out_shape=jax.ShapeDtypeStruct((N, D), jnp.float32),
    )(h, parts, W, b.reshape(1, D), gamma.reshape(1, D), beta.reshape(1, D))
    return out
